# Initial kernel scaffold; baseline (speedup 1.0000x reference)
import jax
import jax.numpy as jnp
from jax.experimental import pallas as pl


def kernel(edge_attr, node_attr, global_attr, edge_index, batch, eW1, eb1, eW2, eb2, eg, ebt, nW1, nb1, nW2, nb2, ng, nbt, gW1, gb1, gW2, gb2, gg, gbt):
    raise NotImplementedError("write your pallas kernel here")


# speedup vs baseline: 1.8776x; 1.8776x over previous
import functools

import jax
import jax.numpy as jnp
from jax import lax
from jax.experimental import pallas as pl
from jax.experimental.pallas import tpu as pltpu

N = 10000
E = 320000
G = 8
DE_IN = 16
DN = 128
DG = 128
L = 128
DE_OUT = 128

BN = 400
BE = 512

_I = False


def _ln_affine(h, g, beta):
    mu = jnp.mean(h, axis=-1, keepdims=True)
    var = jnp.mean((h - mu) ** 2, axis=-1, keepdims=True)
    return (h - mu) * lax.rsqrt(var + 1e-5) * g + beta


def _prep_body(x_ref, oh_ref, ga_ref, wr_ref, ws_ref, wg_ref, eb1_ref,
               nw1a_ref, nw1g_ref, nb1_ref, pr_ref, psp_ref, pnp_ref):
    x = x_ref[...]
    oh = oh_ref[...]
    ga = ga_ref[...]
    pg = jnp.dot(ga, wg_ref[...], preferred_element_type=jnp.float32) + eb1_ref[...]
    qg = jnp.dot(ga, nw1g_ref[...], preferred_element_type=jnp.float32) + nb1_ref[...]
    pr_ref[...] = jnp.dot(x, wr_ref[...], preferred_element_type=jnp.float32)
    psp_ref[...] = (jnp.dot(x, ws_ref[...], preferred_element_type=jnp.float32)
                    + jnp.dot(oh, pg, preferred_element_type=jnp.float32))
    pnp_ref[...] = (jnp.dot(x, nw1a_ref[...], preferred_element_type=jnp.float32)
                    + jnp.dot(oh, qg, preferred_element_type=jnp.float32))


def _prep(node_attr, onehot, ga, wr, ws, wg, eb1, nw1a, nw1g, nb1):
    nb = N // BN
    full = lambda shape: pl.BlockSpec(shape, lambda i: (0, 0))
    blk = lambda width: pl.BlockSpec((BN, width), lambda i: (i, 0))
    return pl.pallas_call(
        _prep_body,
        grid=(nb,),
        in_specs=[blk(DN), blk(G), full((G, DG)), full((DN, L)), full((DN, L)),
                  full((DG, L)), full((1, L)), full((DN, L)), full((DG, L)),
                  full((1, L))],
        out_specs=[blk(L), blk(L), blk(L)],
        out_shape=[jax.ShapeDtypeStruct((N, L), jnp.float32)] * 3,
        interpret=_I,
    )(node_attr, onehot, ga, wr, ws, wg, eb1, nw1a, nw1g, nb1)


def _edge_body(hrs_ref, ea_ref, we_ref, ew2_ref, eb2_ref, eg_ref, ebt_ref,
               out_ref):
    h1 = hrs_ref[...] + jnp.dot(ea_ref[...], we_ref[...],
                                preferred_element_type=jnp.float32)
    h1 = jnp.maximum(h1, 0.0)
    h2 = jnp.dot(h1, ew2_ref[...], preferred_element_type=jnp.float32) + eb2_ref[...]
    h2 = jnp.maximum(h2, 0.0)
    out_ref[...] = _ln_affine(h2, eg_ref[...], ebt_ref[...])


def _edge_mlp(hrs, edge_attr, we, ew2, eb2, eg, ebt):
    nb = E // BE
    full = lambda shape: pl.BlockSpec(shape, lambda i: (0, 0))
    return pl.pallas_call(
        _edge_body,
        grid=(nb,),
        in_specs=[pl.BlockSpec((BE, L), lambda i: (i, 0)),
                  pl.BlockSpec((BE, DE_IN), lambda i: (i, 0)),
                  full((DE_IN, L)), full((L, DE_OUT)), full((1, DE_OUT)),
                  full((1, DE_OUT)), full((1, DE_OUT))],
        out_specs=pl.BlockSpec((BE, DE_OUT), lambda i: (i, 0)),
        out_shape=jax.ShapeDtypeStruct((E, DE_OUT), jnp.float32),
        interpret=_I,
    )(hrs, edge_attr, we, ew2, eb2, eg, ebt)


def _node_body(pnp_ref, scol_ref, srow_ref, deg_ref, oh_ref,
               nw1c_ref, nw1r_ref, nw2_ref, nb2_ref, ng_ref, nbt_ref,
               out_ref, macc_ref):
    h1 = (pnp_ref[...]
          + jnp.dot(scol_ref[...], nw1c_ref[...], preferred_element_type=jnp.float32)
          + jnp.dot(srow_ref[...], nw1r_ref[...], preferred_element_type=jnp.float32))
    h1 = jnp.maximum(h1, 0.0)
    h2 = jnp.dot(h1, nw2_ref[...], preferred_element_type=jnp.float32) + nb2_ref[...]
    h2 = jnp.maximum(h2, 0.0)
    out = _ln_affine(h2, ng_ref[...], nbt_ref[...])
    out_ref[...] = out
    cat = jnp.concatenate(
        [out, srow_ref[...], deg_ref[...], jnp.ones((BN, 16), jnp.float32)],
        axis=1)
    contrib = jnp.dot(oh_ref[...].T, cat, preferred_element_type=jnp.float32)

    @pl.when(pl.program_id(0) == 0)
    def _():
        macc_ref[...] = jnp.zeros_like(macc_ref)

    macc_ref[...] += contrib


def _node_mlp(pnp, scol, srow, deg, onehot, nw1c, nw1r, nw2, nb2, ng, nbt):
    nb = N // BN
    full = lambda shape: pl.BlockSpec(shape, lambda i: (0, 0))
    blk = lambda width: pl.BlockSpec((BN, width), lambda i: (i, 0))
    return pl.pallas_call(
        _node_body,
        grid=(nb,),
        in_specs=[blk(L), blk(DE_OUT), blk(DE_OUT), blk(16), blk(G),
                  full((DE_OUT, L)), full((DE_OUT, L)), full((L, DN)),
                  full((1, DN)), full((1, DN)), full((1, DN))],
        out_specs=[blk(DN), pl.BlockSpec((G, 288), lambda i: (0, 0))],
        out_shape=[jax.ShapeDtypeStruct((N, DN), jnp.float32),
                   jax.ShapeDtypeStruct((G, 288), jnp.float32)],
        interpret=_I,
    )(pnp, scol, srow, deg, onehot, nw1c, nw1r, nw2, nb2, ng, nbt)


def _glob_body(macc_ref, ga_ref, g1n_ref, g1e_ref, g1g_ref, gb1_ref,
               gw2_ref, gb2_ref, gg_ref, gbt_ref, out_ref):
    macc = macc_ref[...]
    n2g = macc[:, 0:128]
    e2g = macc[:, 128:256]
    ecnt = macc[:, 256:257]
    ncnt = macc[:, 272:273]
    n2g = n2g / jnp.maximum(ncnt, 1.0)
    e2g = e2g / jnp.maximum(ecnt, 1.0)
    h1 = (jnp.dot(n2g, g1n_ref[...], preferred_element_type=jnp.float32)
          + jnp.dot(e2g, g1e_ref[...], preferred_element_type=jnp.float32)
          + jnp.dot(ga_ref[...], g1g_ref[...], preferred_element_type=jnp.float32)
          + gb1_ref[...])
    h1 = jnp.maximum(h1, 0.0)
    h2 = jnp.dot(h1, gw2_ref[...], preferred_element_type=jnp.float32) + gb2_ref[...]
    h2 = jnp.maximum(h2, 0.0)
    out_ref[...] = _ln_affine(h2, gg_ref[...], gbt_ref[...])


def _glob_mlp(macc, ga, g1n, g1e, g1g, gb1, gw2, gb2, gg, gbt):
    return pl.pallas_call(
        _glob_body,
        out_shape=jax.ShapeDtypeStruct((G, DG), jnp.float32),
        interpret=_I,
    )(macc, ga, g1n, g1e, g1g, gb1, gw2, gb2, gg, gbt)


def _gather_hrs(pr, psp, row, col):
    return jnp.take(pr, col, axis=0) + jnp.take(psp, row, axis=0)


def _scatter_sums(edge_new, row, col):
    srow = jax.ops.segment_sum(edge_new, row, num_segments=N)
    scol = jax.ops.segment_sum(edge_new, col, num_segments=N)
    deg = jax.ops.segment_sum(jnp.ones((E, 16), jnp.float32), row, num_segments=N)
    return srow, scol, deg


def kernel(edge_attr, node_attr, global_attr, edge_index, batch,
           eW1, eb1, eW2, eb2, eg, ebt,
           nW1, nb1, nW2, nb2, ng, nbt,
           gW1, gb1, gW2, gb2, gg, gbt):
    row = edge_index[0]
    col = edge_index[1]
    onehot = (batch[:, None] == jnp.arange(G, dtype=jnp.int32)[None, :]
              ).astype(jnp.float32)

    wr = eW1[0:128]
    ws = eW1[128:256]
    we = eW1[256:272]
    wg = eW1[272:400]
    nw1a = nW1[0:128]
    nw1g = nW1[128:256]
    nw1c = nW1[256:384]
    nw1r = nW1[384:512]
    g1n = gW1[0:128]
    g1e = gW1[128:256]
    g1g = gW1[256:384]

    r2 = lambda v: v.reshape(1, -1)

    pr, psp, pnp = _prep(node_attr, onehot, global_attr, wr, ws, wg, r2(eb1),
                         nw1a, nw1g, r2(nb1))

    hrs = _gather_hrs(pr, psp, row, col)
    edge_new = _edge_mlp(hrs, edge_attr, we, eW2, r2(eb2), r2(eg), r2(ebt))

    srow, scol, deg = _scatter_sums(edge_new, row, col)

    node_new, macc = _node_mlp(pnp, scol, srow, deg, onehot,
                               nw1c, nw1r, nW2, r2(nb2), r2(ng), r2(nbt))

    global_new = _glob_mlp(macc, global_attr, g1n, g1e, g1g, r2(gb1),
                           gW2, r2(gb2), r2(gg), r2(gbt))

    return (edge_new, node_new, global_new)


# speedup vs baseline: 2.3382x; 1.2453x over previous
import functools

import jax
import jax.numpy as jnp
from jax import lax
from jax.experimental import pallas as pl
from jax.experimental.pallas import tpu as pltpu
from jax.experimental.pallas import tpu_sc as plsc

N = 10000
E = 320000
G = 8
DE_IN = 16
DN = 128
DG = 128
L = 128
DE_OUT = 128

BN = 400
BE = 512

_I = False


def _ln_affine(h, g, beta):
    mu = jnp.mean(h, axis=-1, keepdims=True)
    var = jnp.mean((h - mu) ** 2, axis=-1, keepdims=True)
    return (h - mu) * lax.rsqrt(var + 1e-5) * g + beta


def _prep_body(x_ref, oh_ref, ga_ref, wr_ref, ws_ref, wg_ref, eb1_ref,
               nw1a_ref, nw1g_ref, nb1_ref, pr_ref, psp_ref, pnp_ref):
    x = x_ref[...]
    oh = oh_ref[...]
    ga = ga_ref[...]
    pg = jnp.dot(ga, wg_ref[...], preferred_element_type=jnp.float32) + eb1_ref[...]
    qg = jnp.dot(ga, nw1g_ref[...], preferred_element_type=jnp.float32) + nb1_ref[...]
    pr_ref[...] = jnp.dot(x, wr_ref[...], preferred_element_type=jnp.float32)
    psp_ref[...] = (jnp.dot(x, ws_ref[...], preferred_element_type=jnp.float32)
                    + jnp.dot(oh, pg, preferred_element_type=jnp.float32))
    pnp_ref[...] = (jnp.dot(x, nw1a_ref[...], preferred_element_type=jnp.float32)
                    + jnp.dot(oh, qg, preferred_element_type=jnp.float32))


def _prep(node_attr, onehot, ga, wr, ws, wg, eb1, nw1a, nw1g, nb1):
    nb = N // BN
    full = lambda shape: pl.BlockSpec(shape, lambda i: (0, 0))
    blk = lambda width: pl.BlockSpec((BN, width), lambda i: (i, 0))
    return pl.pallas_call(
        _prep_body,
        grid=(nb,),
        in_specs=[blk(DN), blk(G), full((G, DG)), full((DN, L)), full((DN, L)),
                  full((DG, L)), full((1, L)), full((DN, L)), full((DG, L)),
                  full((1, L))],
        out_specs=[blk(L), blk(L), blk(L)],
        out_shape=[jax.ShapeDtypeStruct((N, L), jnp.float32)] * 3,
        interpret=_I,
    )(node_attr, onehot, ga, wr, ws, wg, eb1, nw1a, nw1g, nb1)


def _edge_body(hrs_ref, ea_ref, we_ref, ew2_ref, eb2_ref, eg_ref, ebt_ref,
               out_ref):
    h1 = hrs_ref[...] + jnp.dot(ea_ref[...], we_ref[...],
                                preferred_element_type=jnp.float32)
    h1 = jnp.maximum(h1, 0.0)
    h2 = jnp.dot(h1, ew2_ref[...], preferred_element_type=jnp.float32) + eb2_ref[...]
    h2 = jnp.maximum(h2, 0.0)
    out_ref[...] = _ln_affine(h2, eg_ref[...], ebt_ref[...])


def _edge_mlp(hrs, edge_attr, we, ew2, eb2, eg, ebt):
    nb = E // BE
    full = lambda shape: pl.BlockSpec(shape, lambda i: (0, 0))
    return pl.pallas_call(
        _edge_body,
        grid=(nb,),
        in_specs=[pl.BlockSpec((BE, L), lambda i: (i, 0)),
                  pl.BlockSpec((BE, DE_IN), lambda i: (i, 0)),
                  full((DE_IN, L)), full((L, DE_OUT)), full((1, DE_OUT)),
                  full((1, DE_OUT)), full((1, DE_OUT))],
        out_specs=pl.BlockSpec((BE, DE_OUT), lambda i: (i, 0)),
        out_shape=jax.ShapeDtypeStruct((E, DE_OUT), jnp.float32),
        interpret=_I,
    )(hrs, edge_attr, we, ew2, eb2, eg, ebt)


def _node_body(pnp_ref, scol_ref, srow_ref, deg_ref, oh_ref,
               nw1c_ref, nw1r_ref, nw2_ref, nb2_ref, ng_ref, nbt_ref,
               out_ref, macc_ref):
    h1 = (pnp_ref[...]
          + jnp.dot(scol_ref[...], nw1c_ref[...], preferred_element_type=jnp.float32)
          + jnp.dot(srow_ref[...], nw1r_ref[...], preferred_element_type=jnp.float32))
    h1 = jnp.maximum(h1, 0.0)
    h2 = jnp.dot(h1, nw2_ref[...], preferred_element_type=jnp.float32) + nb2_ref[...]
    h2 = jnp.maximum(h2, 0.0)
    out = _ln_affine(h2, ng_ref[...], nbt_ref[...])
    out_ref[...] = out
    cat = jnp.concatenate(
        [out, srow_ref[...], deg_ref[...], jnp.ones((BN, 16), jnp.float32)],
        axis=1)
    contrib = jnp.dot(oh_ref[...].T, cat, preferred_element_type=jnp.float32)

    @pl.when(pl.program_id(0) == 0)
    def _():
        macc_ref[...] = jnp.zeros_like(macc_ref)

    macc_ref[...] += contrib


def _node_mlp(pnp, scol, srow, deg, onehot, nw1c, nw1r, nw2, nb2, ng, nbt):
    nb = N // BN
    full = lambda shape: pl.BlockSpec(shape, lambda i: (0, 0))
    blk = lambda width: pl.BlockSpec((BN, width), lambda i: (i, 0))
    return pl.pallas_call(
        _node_body,
        grid=(nb,),
        in_specs=[blk(L), blk(DE_OUT), blk(DE_OUT), blk(16), blk(G),
                  full((DE_OUT, L)), full((DE_OUT, L)), full((L, DN)),
                  full((1, DN)), full((1, DN)), full((1, DN))],
        out_specs=[blk(DN), pl.BlockSpec((G, 288), lambda i: (0, 0))],
        out_shape=[jax.ShapeDtypeStruct((N, DN), jnp.float32),
                   jax.ShapeDtypeStruct((G, 288), jnp.float32)],
        interpret=_I,
    )(pnp, scol, srow, deg, onehot, nw1c, nw1r, nw2, nb2, ng, nbt)


def _glob_body(macc_ref, ga_ref, g1n_ref, g1e_ref, g1g_ref, gb1_ref,
               gw2_ref, gb2_ref, gg_ref, gbt_ref, out_ref):
    macc = macc_ref[...]
    n2g = macc[:, 0:128]
    e2g = macc[:, 128:256]
    ecnt = macc[:, 256:257]
    ncnt = macc[:, 272:273]
    n2g = n2g / jnp.maximum(ncnt, 1.0)
    e2g = e2g / jnp.maximum(ecnt, 1.0)
    h1 = (jnp.dot(n2g, g1n_ref[...], preferred_element_type=jnp.float32)
          + jnp.dot(e2g, g1e_ref[...], preferred_element_type=jnp.float32)
          + jnp.dot(ga_ref[...], g1g_ref[...], preferred_element_type=jnp.float32)
          + gb1_ref[...])
    h1 = jnp.maximum(h1, 0.0)
    h2 = jnp.dot(h1, gw2_ref[...], preferred_element_type=jnp.float32) + gb2_ref[...]
    h2 = jnp.maximum(h2, 0.0)
    out_ref[...] = _ln_affine(h2, gg_ref[...], gbt_ref[...])


def _glob_mlp(macc, ga, g1n, g1e, g1g, gb1, gw2, gb2, gg, gbt):
    return pl.pallas_call(
        _glob_body,
        out_shape=jax.ShapeDtypeStruct((G, DG), jnp.float32),
        interpret=_I,
    )(macc, ga, g1n, g1e, g1g, gb1, gw2, gb2, gg, gbt)


def _gather_hrs(pr, psp, row, col):
    return jnp.take(pr, col, axis=0) + jnp.take(psp, row, axis=0)


_MESH = plsc.VectorSubcoreMesh(core_axis_name="c", subcore_axis_name="s")
NSUB = 16
EPS = E // NSUB
KS = 128
NCS = EPS // KS
TS = EPS - NCS * KS
NPAD = 10240
NPS = NPAD // NSUB


def _sc_scatter(edge_new, eidx, zrow):
    @functools.partial(
        pl.kernel,
        out_type=(jax.ShapeDtypeStruct((NPAD, L), jnp.float32),
                  jax.ShapeDtypeStruct((NPAD, L), jnp.float32)),
        mesh=_MESH,
        scratch_types=[
            pltpu.VMEM((KS, L), jnp.float32),
            pltpu.VMEM((KS,), jnp.int32),
            pltpu.VMEM((TS,), jnp.int32),
            pltpu.VMEM_SHARED((NPAD, L), jnp.float32),
        ],
    )
    def k(edge_hbm, eidx_hbm, zrow_hbm,
          srow_hbm, scol_hbm, ebuf, idxm, idxt, acc):
        cid = lax.axis_index("c")
        sid = lax.axis_index("s")
        r0 = sid * NPS
        pltpu.sync_copy(zrow_hbm, ebuf)

        def zbody(i, carry):
            pltpu.sync_copy(ebuf, acc.at[pl.ds(r0 + i * KS, KS)])
            return carry

        lax.fori_loop(0, NPS // KS, zbody, 0)
        plsc.subcore_barrier()
        base = sid * EPS

        def body(j, carry):
            off = base + j * KS
            pltpu.sync_copy(eidx_hbm.at[pl.ds(cid * E + off, KS)], idxm)
            pltpu.sync_copy(edge_hbm.at[pl.ds(off, KS)], ebuf)
            pltpu.sync_copy(ebuf, acc.at[idxm], add=True)
            return carry

        lax.fori_loop(0, NCS, body, 0)
        offt = base + NCS * KS
        pltpu.sync_copy(eidx_hbm.at[pl.ds(cid * E + offt, TS)], idxt)
        pltpu.sync_copy(edge_hbm.at[pl.ds(offt, TS)], ebuf.at[pl.ds(0, TS)])
        pltpu.sync_copy(ebuf.at[pl.ds(0, TS)], acc.at[idxt], add=True)
        plsc.subcore_barrier()

        def wbody(i, carry):
            rr = r0 + i * KS
            pltpu.sync_copy(acc.at[pl.ds(rr, KS)], ebuf)

            @pl.when(cid == 0)
            def _():
                pltpu.sync_copy(ebuf, srow_hbm.at[pl.ds(rr, KS)])

            @pl.when(cid == 1)
            def _():
                pltpu.sync_copy(ebuf, scol_hbm.at[pl.ds(rr, KS)])
            return carry

        lax.fori_loop(0, NPS // KS, wbody, 0)

    return k(edge_new, eidx, zrow)


def _scatter_sums(edge_new, eidx):
    zrow = jnp.zeros((KS, L), jnp.float32)
    srow, scol = _sc_scatter(edge_new, eidx.reshape(-1), zrow)
    deg = jax.ops.segment_sum(jnp.ones((E, 16), jnp.float32), eidx[0],
                              num_segments=N)
    return srow[:N], scol[:N], deg


def kernel(edge_attr, node_attr, global_attr, edge_index, batch,
           eW1, eb1, eW2, eb2, eg, ebt,
           nW1, nb1, nW2, nb2, ng, nbt,
           gW1, gb1, gW2, gb2, gg, gbt):
    row = edge_index[0]
    col = edge_index[1]
    onehot = (batch[:, None] == jnp.arange(G, dtype=jnp.int32)[None, :]
              ).astype(jnp.float32)

    wr = eW1[0:128]
    ws = eW1[128:256]
    we = eW1[256:272]
    wg = eW1[272:400]
    nw1a = nW1[0:128]
    nw1g = nW1[128:256]
    nw1c = nW1[256:384]
    nw1r = nW1[384:512]
    g1n = gW1[0:128]
    g1e = gW1[128:256]
    g1g = gW1[256:384]

    r2 = lambda v: v.reshape(1, -1)

    pr, psp, pnp = _prep(node_attr, onehot, global_attr, wr, ws, wg, r2(eb1),
                         nw1a, nw1g, r2(nb1))

    hrs = _gather_hrs(pr, psp, row, col)
    edge_new = _edge_mlp(hrs, edge_attr, we, eW2, r2(eb2), r2(eg), r2(ebt))

    srow, scol, deg = _scatter_sums(edge_new, edge_index)

    node_new, macc = _node_mlp(pnp, scol, srow, deg, onehot,
                               nw1c, nw1r, nW2, r2(nb2), r2(ng), r2(nbt))

    global_new = _glob_mlp(macc, global_attr, g1n, g1e, g1g, r2(gb1),
                           gW2, r2(gb2), r2(gg), r2(gbt))

    return (edge_new, node_new, global_new)


# speedup vs baseline: 3.3775x; 1.4445x over previous
import functools

import jax
import jax.numpy as jnp
from jax import lax
from jax.experimental import pallas as pl
from jax.experimental.pallas import tpu as pltpu
from jax.experimental.pallas import tpu_sc as plsc

N = 10000
E = 320000
G = 8
DE_IN = 16
DN = 128
DG = 128
L = 128
DE_OUT = 128

BN = 400
BE = 512

_I = False


def _ln_affine(h, g, beta):
    mu = jnp.mean(h, axis=-1, keepdims=True)
    var = jnp.mean((h - mu) ** 2, axis=-1, keepdims=True)
    return (h - mu) * lax.rsqrt(var + 1e-5) * g + beta


def _prep_body(x_ref, oh_ref, ga_ref, wr_ref, ws_ref, wg_ref, eb1_ref,
               nw1a_ref, nw1g_ref, nb1_ref, pr_ref, psp_ref, pnp_ref):
    x = x_ref[...]
    oh = oh_ref[...]
    ga = ga_ref[...]
    pg = jnp.dot(ga, wg_ref[...], preferred_element_type=jnp.float32) + eb1_ref[...]
    qg = jnp.dot(ga, nw1g_ref[...], preferred_element_type=jnp.float32) + nb1_ref[...]
    pr_ref[...] = jnp.dot(x, wr_ref[...], preferred_element_type=jnp.float32)
    psp_ref[...] = (jnp.dot(x, ws_ref[...], preferred_element_type=jnp.float32)
                    + jnp.dot(oh, pg, preferred_element_type=jnp.float32))
    pnp_ref[...] = (jnp.dot(x, nw1a_ref[...], preferred_element_type=jnp.float32)
                    + jnp.dot(oh, qg, preferred_element_type=jnp.float32))


def _prep(node_attr, onehot, ga, wr, ws, wg, eb1, nw1a, nw1g, nb1):
    nb = N // BN
    full = lambda shape: pl.BlockSpec(shape, lambda i: (0, 0))
    blk = lambda width: pl.BlockSpec((BN, width), lambda i: (i, 0))
    return pl.pallas_call(
        _prep_body,
        grid=(nb,),
        in_specs=[blk(DN), blk(G), full((G, DG)), full((DN, L)), full((DN, L)),
                  full((DG, L)), full((1, L)), full((DN, L)), full((DG, L)),
                  full((1, L))],
        out_specs=[blk(L), blk(L), blk(L)],
        out_shape=[jax.ShapeDtypeStruct((N, L), jnp.float32)] * 3,
        interpret=_I,
    )(node_attr, onehot, ga, wr, ws, wg, eb1, nw1a, nw1g, nb1)


def _edge_body(hr_ref, hs_ref, ea_ref, we_ref, ew2_ref, eb2_ref, eg_ref,
               ebt_ref, out_ref):
    h1 = (hr_ref[...] + hs_ref[...]
          + jnp.dot(ea_ref[...], we_ref[...], preferred_element_type=jnp.float32))
    h1 = jnp.maximum(h1, 0.0)
    h2 = jnp.dot(h1, ew2_ref[...], preferred_element_type=jnp.float32) + eb2_ref[...]
    h2 = jnp.maximum(h2, 0.0)
    out_ref[...] = _ln_affine(h2, eg_ref[...], ebt_ref[...])


def _edge_mlp(hr, hs, edge_attr, we, ew2, eb2, eg, ebt):
    nb = E // BE
    full = lambda shape: pl.BlockSpec(shape, lambda i: (0, 0))
    return pl.pallas_call(
        _edge_body,
        grid=(nb,),
        in_specs=[pl.BlockSpec((BE, L), lambda i: (i, 0)),
                  pl.BlockSpec((BE, L), lambda i: (i, 0)),
                  pl.BlockSpec((BE, DE_IN), lambda i: (i, 0)),
                  full((DE_IN, L)), full((L, DE_OUT)), full((1, DE_OUT)),
                  full((1, DE_OUT)), full((1, DE_OUT))],
        out_specs=pl.BlockSpec((BE, DE_OUT), lambda i: (i, 0)),
        out_shape=jax.ShapeDtypeStruct((E, DE_OUT), jnp.float32),
        interpret=_I,
    )(hr, hs, edge_attr, we, ew2, eb2, eg, ebt)


def _node_body(pnp_ref, scol_ref, srow_ref, deg_ref, oh_ref,
               nw1c_ref, nw1r_ref, nw2_ref, nb2_ref, ng_ref, nbt_ref,
               out_ref, macc_ref):
    h1 = (pnp_ref[...]
          + jnp.dot(scol_ref[...], nw1c_ref[...], preferred_element_type=jnp.float32)
          + jnp.dot(srow_ref[...], nw1r_ref[...], preferred_element_type=jnp.float32))
    h1 = jnp.maximum(h1, 0.0)
    h2 = jnp.dot(h1, nw2_ref[...], preferred_element_type=jnp.float32) + nb2_ref[...]
    h2 = jnp.maximum(h2, 0.0)
    out = _ln_affine(h2, ng_ref[...], nbt_ref[...])
    out_ref[...] = out
    cat = jnp.concatenate(
        [out, srow_ref[...], deg_ref[...], jnp.ones((BN, 16), jnp.float32)],
        axis=1)
    contrib = jnp.dot(oh_ref[...].T, cat, preferred_element_type=jnp.float32)

    @pl.when(pl.program_id(0) == 0)
    def _():
        macc_ref[...] = jnp.zeros_like(macc_ref)

    macc_ref[...] += contrib


def _node_mlp(pnp, scol, srow, deg, onehot, nw1c, nw1r, nw2, nb2, ng, nbt):
    nb = N // BN
    full = lambda shape: pl.BlockSpec(shape, lambda i: (0, 0))
    blk = lambda width: pl.BlockSpec((BN, width), lambda i: (i, 0))
    return pl.pallas_call(
        _node_body,
        grid=(nb,),
        in_specs=[blk(L), blk(DE_OUT), blk(DE_OUT), blk(16), blk(G),
                  full((DE_OUT, L)), full((DE_OUT, L)), full((L, DN)),
                  full((1, DN)), full((1, DN)), full((1, DN))],
        out_specs=[blk(DN), pl.BlockSpec((G, 288), lambda i: (0, 0))],
        out_shape=[jax.ShapeDtypeStruct((N, DN), jnp.float32),
                   jax.ShapeDtypeStruct((G, 288), jnp.float32)],
        interpret=_I,
    )(pnp, scol, srow, deg, onehot, nw1c, nw1r, nw2, nb2, ng, nbt)


def _glob_body(macc_ref, ga_ref, g1n_ref, g1e_ref, g1g_ref, gb1_ref,
               gw2_ref, gb2_ref, gg_ref, gbt_ref, out_ref):
    macc = macc_ref[...]
    n2g = macc[:, 0:128]
    e2g = macc[:, 128:256]
    ecnt = macc[:, 256:257]
    ncnt = macc[:, 272:273]
    n2g = n2g / jnp.maximum(ncnt, 1.0)
    e2g = e2g / jnp.maximum(ecnt, 1.0)
    h1 = (jnp.dot(n2g, g1n_ref[...], preferred_element_type=jnp.float32)
          + jnp.dot(e2g, g1e_ref[...], preferred_element_type=jnp.float32)
          + jnp.dot(ga_ref[...], g1g_ref[...], preferred_element_type=jnp.float32)
          + gb1_ref[...])
    h1 = jnp.maximum(h1, 0.0)
    h2 = jnp.dot(h1, gw2_ref[...], preferred_element_type=jnp.float32) + gb2_ref[...]
    h2 = jnp.maximum(h2, 0.0)
    out_ref[...] = _ln_affine(h2, gg_ref[...], gbt_ref[...])


def _glob_mlp(macc, ga, g1n, g1e, g1g, gb1, gw2, gb2, gg, gbt):
    return pl.pallas_call(
        _glob_body,
        out_shape=jax.ShapeDtypeStruct((G, DG), jnp.float32),
        interpret=_I,
    )(macc, ga, g1n, g1e, g1g, gb1, gw2, gb2, gg, gbt)


def _sc_gather(pr, psp, eidx):
    @functools.partial(
        pl.kernel,
        out_type=(jax.ShapeDtypeStruct((E, L), jnp.float32),
                  jax.ShapeDtypeStruct((E, L), jnp.float32)),
        mesh=_MESH,
        scratch_types=[
            pltpu.VMEM((KS, L), jnp.float32),
            pltpu.VMEM((KS,), jnp.int32),
            pltpu.VMEM((TS,), jnp.int32),
        ],
    )
    def k(pr_hbm, psp_hbm, eidx_hbm, hs_hbm, hr_hbm, ebuf, idxm, idxt):
        cid = lax.axis_index("c")
        sid = lax.axis_index("s")
        base = sid * EPS

        def body(j, carry):
            off = base + j * KS
            pltpu.sync_copy(eidx_hbm.at[pl.ds(cid * E + off, KS)], idxm)

            @pl.when(cid == 0)
            def _():
                pltpu.sync_copy(psp_hbm.at[idxm], ebuf)
                pltpu.sync_copy(ebuf, hs_hbm.at[pl.ds(off, KS)])

            @pl.when(cid == 1)
            def _():
                pltpu.sync_copy(pr_hbm.at[idxm], ebuf)
                pltpu.sync_copy(ebuf, hr_hbm.at[pl.ds(off, KS)])
            return carry

        lax.fori_loop(0, NCS, body, 0)
        offt = base + NCS * KS
        pltpu.sync_copy(eidx_hbm.at[pl.ds(cid * E + offt, TS)], idxt)

        @pl.when(cid == 0)
        def _():
            pltpu.sync_copy(psp_hbm.at[idxt], ebuf.at[pl.ds(0, TS)])
            pltpu.sync_copy(ebuf.at[pl.ds(0, TS)], hs_hbm.at[pl.ds(offt, TS)])

        @pl.when(cid == 1)
        def _():
            pltpu.sync_copy(pr_hbm.at[idxt], ebuf.at[pl.ds(0, TS)])
            pltpu.sync_copy(ebuf.at[pl.ds(0, TS)], hr_hbm.at[pl.ds(offt, TS)])

    return k(pr, psp, eidx)


_MESH = plsc.VectorSubcoreMesh(core_axis_name="c", subcore_axis_name="s")
NSUB = 16
EPS = E // NSUB
KS = 128
NCS = EPS // KS
TS = EPS - NCS * KS
NPAD = 10240
NPS = NPAD // NSUB


def _sc_scatter(edge_new, eidx, zrow):
    @functools.partial(
        pl.kernel,
        out_type=(jax.ShapeDtypeStruct((NPAD, L), jnp.float32),
                  jax.ShapeDtypeStruct((NPAD, L), jnp.float32)),
        mesh=_MESH,
        scratch_types=[
            pltpu.VMEM((KS, L), jnp.float32),
            pltpu.VMEM((KS,), jnp.int32),
            pltpu.VMEM((TS,), jnp.int32),
            pltpu.VMEM_SHARED((NPAD, L), jnp.float32),
        ],
    )
    def k(edge_hbm, eidx_hbm, zrow_hbm,
          srow_hbm, scol_hbm, ebuf, idxm, idxt, acc):
        cid = lax.axis_index("c")
        sid = lax.axis_index("s")
        r0 = sid * NPS
        pltpu.sync_copy(zrow_hbm, ebuf)

        def zbody(i, carry):
            pltpu.sync_copy(ebuf, acc.at[pl.ds(r0 + i * KS, KS)])
            return carry

        lax.fori_loop(0, NPS // KS, zbody, 0)
        plsc.subcore_barrier()
        base = sid * EPS

        def body(j, carry):
            off = base + j * KS
            pltpu.sync_copy(eidx_hbm.at[pl.ds(cid * E + off, KS)], idxm)
            pltpu.sync_copy(edge_hbm.at[pl.ds(off, KS)], ebuf)
            pltpu.sync_copy(ebuf, acc.at[idxm], add=True)
            return carry

        lax.fori_loop(0, NCS, body, 0)
        offt = base + NCS * KS
        pltpu.sync_copy(eidx_hbm.at[pl.ds(cid * E + offt, TS)], idxt)
        pltpu.sync_copy(edge_hbm.at[pl.ds(offt, TS)], ebuf.at[pl.ds(0, TS)])
        pltpu.sync_copy(ebuf.at[pl.ds(0, TS)], acc.at[idxt], add=True)
        plsc.subcore_barrier()

        def wbody(i, carry):
            rr = r0 + i * KS
            pltpu.sync_copy(acc.at[pl.ds(rr, KS)], ebuf)

            @pl.when(cid == 0)
            def _():
                pltpu.sync_copy(ebuf, srow_hbm.at[pl.ds(rr, KS)])

            @pl.when(cid == 1)
            def _():
                pltpu.sync_copy(ebuf, scol_hbm.at[pl.ds(rr, KS)])
            return carry

        lax.fori_loop(0, NPS // KS, wbody, 0)

    return k(edge_new, eidx, zrow)


def _scatter_sums(edge_new, eidx, row):
    zrow = jnp.zeros((KS, L), jnp.float32)
    srow, scol = _sc_scatter(edge_new, eidx, zrow)
    deg = jax.ops.segment_sum(jnp.ones((E, 16), jnp.float32), row,
                              num_segments=N)
    return srow[:N], scol[:N], deg


def kernel(edge_attr, node_attr, global_attr, edge_index, batch,
           eW1, eb1, eW2, eb2, eg, ebt,
           nW1, nb1, nW2, nb2, ng, nbt,
           gW1, gb1, gW2, gb2, gg, gbt):
    row = edge_index[0]
    col = edge_index[1]
    onehot = (batch[:, None] == jnp.arange(G, dtype=jnp.int32)[None, :]
              ).astype(jnp.float32)

    wr = eW1[0:128]
    ws = eW1[128:256]
    we = eW1[256:272]
    wg = eW1[272:400]
    nw1a = nW1[0:128]
    nw1g = nW1[128:256]
    nw1c = nW1[256:384]
    nw1r = nW1[384:512]
    g1n = gW1[0:128]
    g1e = gW1[128:256]
    g1g = gW1[256:384]

    r2 = lambda v: v.reshape(1, -1)

    pr, psp, pnp = _prep(node_attr, onehot, global_attr, wr, ws, wg, r2(eb1),
                         nw1a, nw1g, r2(nb1))

    eflat = edge_index.reshape(-1)
    hs, hr = _sc_gather(pr, psp, eflat)
    edge_new = _edge_mlp(hr, hs, edge_attr, we, eW2, r2(eb2), r2(eg), r2(ebt))

    srow, scol, deg = _scatter_sums(edge_new, eflat, row)

    node_new, macc = _node_mlp(pnp, scol, srow, deg, onehot,
                               nw1c, nw1r, nW2, r2(nb2), r2(ng), r2(nbt))

    global_new = _glob_mlp(macc, global_attr, g1n, g1e, g1g, r2(gb1),
                           gW2, r2(gb2), r2(gg), r2(gbt))

    return (edge_new, node_new, global_new)


# speedup vs baseline: 4.8157x; 1.4258x over previous
import functools

import jax
import jax.numpy as jnp
from jax import lax
from jax.experimental import pallas as pl
from jax.experimental.pallas import tpu as pltpu
from jax.experimental.pallas import tpu_sc as plsc

N = 10000
E = 320000
G = 8
DE_IN = 16
DN = 128
DG = 128
L = 128
DE_OUT = 128

BN = 400
BE = 512

_I = False


def _ln_affine(h, g, beta):
    mu = jnp.mean(h, axis=-1, keepdims=True)
    var = jnp.mean((h - mu) ** 2, axis=-1, keepdims=True)
    return (h - mu) * lax.rsqrt(var + 1e-5) * g + beta


def _prep_body(x_ref, oh_ref, ga_ref, wr_ref, ws_ref, wg_ref, eb1_ref,
               nw1a_ref, nw1g_ref, nb1_ref, pr_ref, psp_ref, pnp_ref):
    x = x_ref[...]
    oh = oh_ref[...]
    ga = ga_ref[...]
    pg = jnp.dot(ga, wg_ref[...], preferred_element_type=jnp.float32) + eb1_ref[...]
    qg = jnp.dot(ga, nw1g_ref[...], preferred_element_type=jnp.float32) + nb1_ref[...]
    pr_ref[...] = jnp.dot(x, wr_ref[...], preferred_element_type=jnp.float32)
    psp_ref[...] = (jnp.dot(x, ws_ref[...], preferred_element_type=jnp.float32)
                    + jnp.dot(oh, pg, preferred_element_type=jnp.float32))
    pnp_ref[...] = (jnp.dot(x, nw1a_ref[...], preferred_element_type=jnp.float32)
                    + jnp.dot(oh, qg, preferred_element_type=jnp.float32))


def _prep(node_attr, onehot, ga, wr, ws, wg, eb1, nw1a, nw1g, nb1):
    nb = N // BN
    full = lambda shape: pl.BlockSpec(shape, lambda i: (0, 0))
    blk = lambda width: pl.BlockSpec((BN, width), lambda i: (i, 0))
    return pl.pallas_call(
        _prep_body,
        grid=(nb,),
        in_specs=[blk(DN), blk(G), full((G, DG)), full((DN, L)), full((DN, L)),
                  full((DG, L)), full((1, L)), full((DN, L)), full((DG, L)),
                  full((1, L))],
        out_specs=[blk(L), blk(L), blk(L)],
        out_shape=[jax.ShapeDtypeStruct((N, L), jnp.float32)] * 3,
        interpret=_I,
    )(node_attr, onehot, ga, wr, ws, wg, eb1, nw1a, nw1g, nb1)


def _edge_body(hr_ref, hs_ref, ea_ref, we_ref, ew2_ref, eb2_ref, eg_ref,
               ebt_ref, out_ref):
    h1 = (hr_ref[...] + hs_ref[...]
          + jnp.dot(ea_ref[...], we_ref[...], preferred_element_type=jnp.float32))
    h1 = jnp.maximum(h1, 0.0)
    h2 = jnp.dot(h1, ew2_ref[...], preferred_element_type=jnp.float32) + eb2_ref[...]
    h2 = jnp.maximum(h2, 0.0)
    out_ref[...] = _ln_affine(h2, eg_ref[...], ebt_ref[...])


def _edge_mlp(hr, hs, edge_attr, we, ew2, eb2, eg, ebt):
    nb = E // BE
    full = lambda shape: pl.BlockSpec(shape, lambda i: (0, 0))
    return pl.pallas_call(
        _edge_body,
        grid=(nb,),
        in_specs=[pl.BlockSpec((BE, L), lambda i: (i, 0)),
                  pl.BlockSpec((BE, L), lambda i: (i, 0)),
                  pl.BlockSpec((BE, DE_IN), lambda i: (i, 0)),
                  full((DE_IN, L)), full((L, DE_OUT)), full((1, DE_OUT)),
                  full((1, DE_OUT)), full((1, DE_OUT))],
        out_specs=pl.BlockSpec((BE, DE_OUT), lambda i: (i, 0)),
        out_shape=jax.ShapeDtypeStruct((E, DE_OUT), jnp.float32),
        interpret=_I,
    )(hr, hs, edge_attr, we, ew2, eb2, eg, ebt)


def _node_body(pnp_ref, scol_ref, srow_ref, deg_ref, oh_ref,
               nw1c_ref, nw1r_ref, nw2_ref, nb2_ref, ng_ref, nbt_ref,
               out_ref, macc_ref):
    h1 = (pnp_ref[...]
          + jnp.dot(scol_ref[...], nw1c_ref[...], preferred_element_type=jnp.float32)
          + jnp.dot(srow_ref[...], nw1r_ref[...], preferred_element_type=jnp.float32))
    h1 = jnp.maximum(h1, 0.0)
    h2 = jnp.dot(h1, nw2_ref[...], preferred_element_type=jnp.float32) + nb2_ref[...]
    h2 = jnp.maximum(h2, 0.0)
    out = _ln_affine(h2, ng_ref[...], nbt_ref[...])
    out_ref[...] = out
    cat = jnp.concatenate(
        [out, srow_ref[...], deg_ref[...], jnp.ones((BN, 16), jnp.float32)],
        axis=1)
    contrib = jnp.dot(oh_ref[...].T, cat, preferred_element_type=jnp.float32)

    @pl.when(pl.program_id(0) == 0)
    def _():
        macc_ref[...] = jnp.zeros_like(macc_ref)

    macc_ref[...] += contrib


def _node_mlp(pnp, scol, srow, deg, onehot, nw1c, nw1r, nw2, nb2, ng, nbt):
    nb = N // BN
    full = lambda shape: pl.BlockSpec(shape, lambda i: (0, 0))
    blk = lambda width: pl.BlockSpec((BN, width), lambda i: (i, 0))
    return pl.pallas_call(
        _node_body,
        grid=(nb,),
        in_specs=[blk(L), blk(DE_OUT), blk(DE_OUT), blk(16), blk(G),
                  full((DE_OUT, L)), full((DE_OUT, L)), full((L, DN)),
                  full((1, DN)), full((1, DN)), full((1, DN))],
        out_specs=[blk(DN), pl.BlockSpec((G, 288), lambda i: (0, 0))],
        out_shape=[jax.ShapeDtypeStruct((N, DN), jnp.float32),
                   jax.ShapeDtypeStruct((G, 288), jnp.float32)],
        interpret=_I,
    )(pnp, scol, srow, deg, onehot, nw1c, nw1r, nw2, nb2, ng, nbt)


def _glob_body(macc_ref, ga_ref, g1n_ref, g1e_ref, g1g_ref, gb1_ref,
               gw2_ref, gb2_ref, gg_ref, gbt_ref, out_ref):
    macc = macc_ref[...]
    n2g = macc[:, 0:128]
    e2g = macc[:, 128:256]
    ecnt = macc[:, 256:257]
    ncnt = macc[:, 272:273]
    n2g = n2g / jnp.maximum(ncnt, 1.0)
    e2g = e2g / jnp.maximum(ecnt, 1.0)
    h1 = (jnp.dot(n2g, g1n_ref[...], preferred_element_type=jnp.float32)
          + jnp.dot(e2g, g1e_ref[...], preferred_element_type=jnp.float32)
          + jnp.dot(ga_ref[...], g1g_ref[...], preferred_element_type=jnp.float32)
          + gb1_ref[...])
    h1 = jnp.maximum(h1, 0.0)
    h2 = jnp.dot(h1, gw2_ref[...], preferred_element_type=jnp.float32) + gb2_ref[...]
    h2 = jnp.maximum(h2, 0.0)
    out_ref[...] = _ln_affine(h2, gg_ref[...], gbt_ref[...])


def _glob_mlp(macc, ga, g1n, g1e, g1g, gb1, gw2, gb2, gg, gbt):
    return pl.pallas_call(
        _glob_body,
        out_shape=jax.ShapeDtypeStruct((G, DG), jnp.float32),
        interpret=_I,
    )(macc, ga, g1n, g1e, g1g, gb1, gw2, gb2, gg, gbt)


def _sc_gather(pr, psp, eidx):
    @functools.partial(
        pl.kernel,
        out_type=(jax.ShapeDtypeStruct((E, L), jnp.float32),
                  jax.ShapeDtypeStruct((E, L), jnp.float32)),
        mesh=_MESH,
        scratch_types=[
            pltpu.VMEM((KS, L), jnp.float32),
            pltpu.VMEM((KS,), jnp.int32),
            pltpu.VMEM((TS,), jnp.int32),
        ],
    )
    def k(pr_hbm, psp_hbm, eidx_hbm, hs_hbm, hr_hbm, ebuf, idxm, idxt):
        cid = lax.axis_index("c")
        sid = lax.axis_index("s")
        base = sid * EPS

        def body(j, carry):
            off = base + j * KS
            pltpu.sync_copy(eidx_hbm.at[pl.ds(cid * E + off, KS)], idxm)

            @pl.when(cid == 0)
            def _():
                pltpu.sync_copy(psp_hbm.at[idxm], ebuf)
                pltpu.sync_copy(ebuf, hs_hbm.at[pl.ds(off, KS)])

            @pl.when(cid == 1)
            def _():
                pltpu.sync_copy(pr_hbm.at[idxm], ebuf)
                pltpu.sync_copy(ebuf, hr_hbm.at[pl.ds(off, KS)])
            return carry

        lax.fori_loop(0, NCS, body, 0)
        offt = base + NCS * KS
        pltpu.sync_copy(eidx_hbm.at[pl.ds(cid * E + offt, TS)], idxt)

        @pl.when(cid == 0)
        def _():
            pltpu.sync_copy(psp_hbm.at[idxt], ebuf.at[pl.ds(0, TS)])
            pltpu.sync_copy(ebuf.at[pl.ds(0, TS)], hs_hbm.at[pl.ds(offt, TS)])

        @pl.when(cid == 1)
        def _():
            pltpu.sync_copy(pr_hbm.at[idxt], ebuf.at[pl.ds(0, TS)])
            pltpu.sync_copy(ebuf.at[pl.ds(0, TS)], hr_hbm.at[pl.ds(offt, TS)])

    return k(pr, psp, eidx)


_MESH = plsc.VectorSubcoreMesh(core_axis_name="c", subcore_axis_name="s")
NSUB = 16
EPS = E // NSUB
KS = 128
NCS = EPS // KS
TS = EPS - NCS * KS
NPAD = 10240
NPS = NPAD // NSUB


NP128 = NPAD // 128


def _sc_scatter(edge_new, eidx, zrow, iota80):
    @functools.partial(
        pl.kernel,
        out_type=(jax.ShapeDtypeStruct((NPAD, L), jnp.float32),
                  jax.ShapeDtypeStruct((NPAD, L), jnp.float32),
                  jax.ShapeDtypeStruct((NP128, 128), jnp.float32)),
        mesh=_MESH,
        scratch_types=[
            pltpu.VMEM((KS, L), jnp.float32),
            pltpu.VMEM((KS,), jnp.int32),
            pltpu.VMEM((TS,), jnp.int32),
            pltpu.VMEM((NP128, 128), jnp.float32),
            pltpu.VMEM((NP128,), jnp.int32),
            pltpu.VMEM_SHARED((NPAD, L), jnp.float32),
            pltpu.VMEM_SHARED((NP128, 128), jnp.float32),
        ],
        compiler_params=pltpu.CompilerParams(needs_layout_passes=False),
    )
    def k(edge_hbm, eidx_hbm, zrow_hbm, iota_hbm,
          srow_hbm, scol_hbm, degp_hbm, ebuf, idxm, idxt, pd, idx80,
          acc, dacc):
        cid = lax.axis_index("c")
        sid = lax.axis_index("s")
        r0 = sid * NPS
        mask127 = jnp.full((16,), 127, jnp.int32)
        one16 = jnp.full((16,), 1.0, jnp.float32)

        def _deg_accum(idx_ref, nrows):
            for i in range(nrows // 16):
                v = idx_ref[pl.ds(i * 16, 16)]
                hi = lax.shift_right_logical(v, 7)
                lo = lax.bitwise_and(v, mask127)
                plsc.addupdate_scatter(pd, [hi, lo], one16)

        pltpu.sync_copy(zrow_hbm, ebuf)

        def zbody(i, carry):
            pltpu.sync_copy(ebuf, acc.at[pl.ds(r0 + i * KS, KS)])
            return carry

        lax.fori_loop(0, NPS // KS, zbody, 0)

        @pl.when(cid == 0)
        def _():
            pltpu.sync_copy(zrow_hbm.at[pl.ds(0, NP128)], pd)
            pltpu.sync_copy(iota_hbm, idx80)

            @pl.when(sid == 0)
            def _():
                pltpu.sync_copy(zrow_hbm.at[pl.ds(0, NP128)],
                                dacc.at[pl.ds(0, NP128)])

        plsc.subcore_barrier()
        base = sid * EPS

        def body(j, carry):
            off = base + j * KS
            pltpu.sync_copy(eidx_hbm.at[pl.ds(cid * E + off, KS)], idxm)
            pltpu.sync_copy(edge_hbm.at[pl.ds(off, KS)], ebuf)
            pltpu.sync_copy(ebuf, acc.at[idxm], add=True)

            @pl.when(cid == 0)
            def _():
                _deg_accum(idxm, KS)
            return carry

        lax.fori_loop(0, NCS, body, 0)
        offt = base + NCS * KS
        pltpu.sync_copy(eidx_hbm.at[pl.ds(cid * E + offt, TS)], idxt)
        pltpu.sync_copy(edge_hbm.at[pl.ds(offt, TS)], ebuf.at[pl.ds(0, TS)])
        pltpu.sync_copy(ebuf.at[pl.ds(0, TS)], acc.at[idxt], add=True)

        @pl.when(cid == 0)
        def _():
            _deg_accum(idxt, TS)
            pltpu.sync_copy(pd, dacc.at[idx80], add=True)

        plsc.subcore_barrier()

        def wbody(i, carry):
            rr = r0 + i * KS
            pltpu.sync_copy(acc.at[pl.ds(rr, KS)], ebuf)

            @pl.when(cid == 0)
            def _():
                pltpu.sync_copy(ebuf, srow_hbm.at[pl.ds(rr, KS)])

            @pl.when(cid == 1)
            def _():
                pltpu.sync_copy(ebuf, scol_hbm.at[pl.ds(rr, KS)])
            return carry

        lax.fori_loop(0, NPS // KS, wbody, 0)

        @pl.when(jnp.logical_and(cid == 0, sid == 0))
        def _():
            pltpu.sync_copy(dacc.at[pl.ds(0, NP128)], ebuf.at[pl.ds(0, NP128)])
            pltpu.sync_copy(ebuf.at[pl.ds(0, NP128)], degp_hbm)

    return k(edge_new, eidx, zrow, iota80)


def _scatter_sums(edge_new, eidx):
    zrow = jnp.zeros((KS, L), jnp.float32)
    iota80 = jnp.arange(NP128, dtype=jnp.int32)
    srow, scol, degp = _sc_scatter(edge_new, eidx, zrow, iota80)
    deg = jnp.broadcast_to(degp.reshape(-1)[:N, None], (N, 16))
    return srow[:N], scol[:N], deg


def kernel(edge_attr, node_attr, global_attr, edge_index, batch,
           eW1, eb1, eW2, eb2, eg, ebt,
           nW1, nb1, nW2, nb2, ng, nbt,
           gW1, gb1, gW2, gb2, gg, gbt):
    row = edge_index[0]
    col = edge_index[1]
    onehot = (batch[:, None] == jnp.arange(G, dtype=jnp.int32)[None, :]
              ).astype(jnp.float32)

    wr = eW1[0:128]
    ws = eW1[128:256]
    we = eW1[256:272]
    wg = eW1[272:400]
    nw1a = nW1[0:128]
    nw1g = nW1[128:256]
    nw1c = nW1[256:384]
    nw1r = nW1[384:512]
    g1n = gW1[0:128]
    g1e = gW1[128:256]
    g1g = gW1[256:384]

    r2 = lambda v: v.reshape(1, -1)

    pr, psp, pnp = _prep(node_attr, onehot, global_attr, wr, ws, wg, r2(eb1),
                         nw1a, nw1g, r2(nb1))

    eflat = edge_index.reshape(-1)
    hs, hr = _sc_gather(pr, psp, eflat)
    edge_new = _edge_mlp(hr, hs, edge_attr, we, eW2, r2(eb2), r2(eg), r2(ebt))

    srow, scol, deg = _scatter_sums(edge_new, eflat)

    node_new, macc = _node_mlp(pnp, scol, srow, deg, onehot,
                               nw1c, nw1r, nW2, r2(nb2), r2(ng), r2(nbt))

    global_new = _glob_mlp(macc, global_attr, g1n, g1e, g1g, r2(gb1),
                           gW2, r2(gb2), r2(gg), r2(gbt))

    return (edge_new, node_new, global_new)
